# Initial kernel scaffold; baseline (speedup 1.0000x reference)
import jax
import jax.numpy as jnp
from jax.experimental import pallas as pl


def kernel(x, edge_index, W1a, b1a, W1b, b1b, W2a, b2a, W2b, b2b, Wl, bl):
    raise NotImplementedError("write your pallas kernel here")


# speedup vs baseline: 2.3103x; 2.3103x over previous
import functools

import jax
import jax.numpy as jnp
from jax import lax
from jax.experimental import pallas as pl
from jax.experimental.pallas import tpu as pltpu
from jax.experimental.pallas import tpu_sc as plsc

N = 10000
D = 128
E = 320000
OUT = 2

NC = 2
NS = 16
NW = NC * NS

K = 128
EPW = 10240
CHUNKS = EPW // K
E_PAD = EPW * NW
NPAD = 10112
ROWS_PER_SUB = NPAD // NS

_sc_mesh = plsc.VectorSubcoreMesh(core_axis_name="c", subcore_axis_name="s")


def _agg_body(table, src_i, dst_i, zero_hbm, out, acc, src_v, dst_v, rows_v, sem):
    c = lax.axis_index("c")
    s = lax.axis_index("s")
    wid = s * NC + c
    pltpu.sync_copy(zero_hbm, acc.at[pl.ds(s * ROWS_PER_SUB, ROWS_PER_SUB)])
    plsc.subcore_barrier()
    base = wid * EPW

    def chunk(i, carry):
        off = base + i * K
        pltpu.sync_copy(src_i.at[pl.ds(off, K)], src_v)
        pltpu.sync_copy(dst_i.at[pl.ds(off, K)], dst_v)
        pltpu.async_copy(table.at[src_v], rows_v, sem).wait()
        pltpu.sync_copy(rows_v, acc.at[dst_v], add=True)
        return carry

    lax.fori_loop(0, CHUNKS, chunk, 0)
    plsc.subcore_barrier()
    pltpu.sync_copy(acc.at[pl.ds(s * ROWS_PER_SUB, ROWS_PER_SUB)],
                    out.at[c, pl.ds(s * ROWS_PER_SUB, ROWS_PER_SUB)])


_agg_call = functools.partial(
    pl.kernel,
    _agg_body,
    out_type=jax.ShapeDtypeStruct((NC, NPAD, D), jnp.float32),
    mesh=_sc_mesh,
    scratch_types=[
        pltpu.VMEM_SHARED((NPAD, D), jnp.float32),
        pltpu.VMEM((K,), jnp.int32),
        pltpu.VMEM((K,), jnp.int32),
        pltpu.VMEM((K, D), jnp.float32),
        pltpu.SemaphoreType.DMA,
    ],
)()


ROWS_TC = 1000


def _mlp1_body(x_ref, a0_ref, a1_ref, wa_ref, ba_ref, wb_ref, bb_ref, o_ref):
    h = x_ref[...] + a0_ref[...] + a1_ref[...]
    t = jnp.dot(h, wa_ref[...], preferred_element_type=jnp.float32) + ba_ref[...]
    t = jnp.maximum(t, 0.0)
    u = jnp.dot(t, wb_ref[...], preferred_element_type=jnp.float32) + bb_ref[...]
    o_ref[...] = jnp.maximum(u, 0.0)


def _mlp2_body(x_ref, a0_ref, a1_ref, wa_ref, ba_ref, wb_ref, bb_ref,
               wl_ref, bl_ref, o_ref):
    h = x_ref[...] + a0_ref[...] + a1_ref[...]
    t = jnp.dot(h, wa_ref[...], preferred_element_type=jnp.float32) + ba_ref[...]
    t = jnp.maximum(t, 0.0)
    u = jnp.dot(t, wb_ref[...], preferred_element_type=jnp.float32) + bb_ref[...]
    u = jnp.maximum(u, 0.0)
    o_ref[...] = jnp.dot(u, wl_ref[...], preferred_element_type=jnp.float32) + bl_ref[...]


def _row_spec():
    return pl.BlockSpec((ROWS_TC, D), lambda i: (i, 0))


def _full_spec(shape):
    return pl.BlockSpec(shape, lambda i: (0,) * len(shape))


def _mlp1(x, a0, a1, wa, ba, wb, bb):
    return pl.pallas_call(
        _mlp1_body,
        grid=(N // ROWS_TC,),
        in_specs=[_row_spec(), _row_spec(), _row_spec(),
                  _full_spec((D, D)), _full_spec((1, D)),
                  _full_spec((D, D)), _full_spec((1, D))],
        out_specs=_row_spec(),
        out_shape=jax.ShapeDtypeStruct((N, D), jnp.float32),
    )(x, a0, a1, wa, ba.reshape(1, D), wb, bb.reshape(1, D))


def _mlp2(x, a0, a1, wa, ba, wb, bb, wl_pad, bl_pad):
    return pl.pallas_call(
        _mlp2_body,
        grid=(N // ROWS_TC,),
        in_specs=[_row_spec(), _row_spec(), _row_spec(),
                  _full_spec((D, D)), _full_spec((1, D)),
                  _full_spec((D, D)), _full_spec((1, D)),
                  _full_spec((D, D)), _full_spec((1, D))],
        out_specs=_row_spec(),
        out_shape=jax.ShapeDtypeStruct((N, D), jnp.float32),
    )(x, a0, a1, wa, ba.reshape(1, D), wb, bb.reshape(1, D), wl_pad, bl_pad)


def kernel(x, edge_index, W1a, b1a, W1b, b1b, W2a, b2a, W2b, b2b, Wl, bl):
    src = edge_index[0]
    dst = edge_index[1]
    pad = E_PAD - E
    src_p = jnp.concatenate([src, jnp.zeros((pad,), jnp.int32)])
    dst_p = jnp.concatenate([dst, jnp.full((pad,), N, jnp.int32)])
    zero = jnp.zeros((ROWS_PER_SUB, D), jnp.float32)

    parts1 = _agg_call(x, src_p, dst_p, zero)
    h1 = _mlp1(x, parts1[0, :N], parts1[1, :N], W1a, b1a, W1b, b1b)

    parts2 = _agg_call(h1, src_p, dst_p, zero)
    wl_pad = jnp.zeros((D, D), jnp.float32).at[:, :OUT].set(Wl)
    bl_pad = jnp.zeros((1, D), jnp.float32).at[0, :OUT].set(bl)
    out_full = _mlp2(h1, parts2[0, :N], parts2[1, :N], W2a, b2a, W2b, b2b,
                     wl_pad, bl_pad)
    return out_full[:, :OUT]


# speedup vs baseline: 2.3126x; 1.0010x over previous
import functools

import jax
import jax.numpy as jnp
from jax import lax
from jax.experimental import pallas as pl
from jax.experimental.pallas import tpu as pltpu
from jax.experimental.pallas import tpu_sc as plsc

N = 10000
D = 128
E = 320000
OUT = 2

NC = 2
NS = 16
NW = NC * NS

K = 128
EPW = 10240
CHUNKS = EPW // K
E_PAD = EPW * NW
NPAD = 10112
ROWS_PER_SUB = NPAD // NS

_sc_mesh = plsc.VectorSubcoreMesh(core_axis_name="c", subcore_axis_name="s")


def _agg_body(table, src_i, dst_i, zero_hbm, out, acc, src_v, dst_v, rows_v, sem):
    c = lax.axis_index("c")
    s = lax.axis_index("s")
    wid = s * NC + c
    pltpu.sync_copy(zero_hbm, acc.at[pl.ds(s * ROWS_PER_SUB, ROWS_PER_SUB)])
    plsc.subcore_barrier()
    base = wid * EPW

    def chunk(i, carry):
        off = base + i * K
        pltpu.sync_copy(src_i.at[pl.ds(off, K)], src_v)
        pltpu.sync_copy(dst_i.at[pl.ds(off, K)], dst_v)
        pltpu.async_copy(table.at[src_v], rows_v, sem).wait()
        pltpu.sync_copy(rows_v, acc.at[dst_v], add=True)
        return carry

    lax.fori_loop(0, CHUNKS, chunk, 0)
    plsc.subcore_barrier()
    pltpu.sync_copy(acc.at[pl.ds(s * ROWS_PER_SUB, ROWS_PER_SUB)],
                    out.at[c, pl.ds(s * ROWS_PER_SUB, ROWS_PER_SUB)])


_agg_call = functools.partial(
    pl.kernel,
    _agg_body,
    out_type=jax.ShapeDtypeStruct((NC, NPAD, D), jnp.float32),
    mesh=_sc_mesh,
    scratch_types=[
        pltpu.VMEM_SHARED((NPAD, D), jnp.float32),
        pltpu.VMEM((K,), jnp.int32),
        pltpu.VMEM((K,), jnp.int32),
        pltpu.VMEM((K, D), jnp.float32),
        pltpu.SemaphoreType.DMA,
    ],
)()


ROWS_TC = 1000


def _mlp1_body(x_ref, a0_ref, a1_ref, wa_ref, ba_ref, wb_ref, bb_ref, o_ref):
    h = x_ref[...] + a0_ref[...] + a1_ref[...]
    t = jnp.dot(h, wa_ref[...], preferred_element_type=jnp.float32) + ba_ref[...]
    t = jnp.maximum(t, 0.0)
    u = jnp.dot(t, wb_ref[...], preferred_element_type=jnp.float32) + bb_ref[...]
    o_ref[...] = jnp.maximum(u, 0.0)


def _mlp2_body(x_ref, a0_ref, a1_ref, wa_ref, ba_ref, wb_ref, bb_ref,
               wl_ref, bl_ref, o_ref):
    h = x_ref[...] + a0_ref[...] + a1_ref[...]
    t = jnp.dot(h, wa_ref[...], preferred_element_type=jnp.float32) + ba_ref[...]
    t = jnp.maximum(t, 0.0)
    u = jnp.dot(t, wb_ref[...], preferred_element_type=jnp.float32) + bb_ref[...]
    u = jnp.maximum(u, 0.0)
    o_ref[...] = jnp.dot(u, wl_ref[...], preferred_element_type=jnp.float32) + bl_ref[...]


def _row_spec():
    return pl.BlockSpec((ROWS_TC, D), lambda i: (i, 0))


def _full_spec(shape):
    return pl.BlockSpec(shape, lambda i: (0,) * len(shape))


def _mlp1(x, a0, a1, wa, ba, wb, bb):
    return pl.pallas_call(
        _mlp1_body,
        grid=(N // ROWS_TC,),
        in_specs=[_row_spec(), _row_spec(), _row_spec(),
                  _full_spec((D, D)), _full_spec((1, D)),
                  _full_spec((D, D)), _full_spec((1, D))],
        out_specs=_row_spec(),
        out_shape=jax.ShapeDtypeStruct((N, D), jnp.float32),
    )(x, a0, a1, wa, ba.reshape(1, D), wb, bb.reshape(1, D))


def _mlp2(x, a0, a1, wa, ba, wb, bb, wl_pad, bl_pad):
    return pl.pallas_call(
        _mlp2_body,
        grid=(N // ROWS_TC,),
        in_specs=[_row_spec(), _row_spec(), _row_spec(),
                  _full_spec((D, D)), _full_spec((1, D)),
                  _full_spec((D, D)), _full_spec((1, D)),
                  _full_spec((D, D)), _full_spec((1, D))],
        out_specs=_row_spec(),
        out_shape=jax.ShapeDtypeStruct((N, D), jnp.float32),
    )(x, a0, a1, wa, ba.reshape(1, D), wb, bb.reshape(1, D), wl_pad, bl_pad)


def kernel(x, edge_index, W1a, b1a, W1b, b1b, W2a, b2a, W2b, b2b, Wl, bl):
    src = edge_index[0]
    dst = edge_index[1]
    pad = E_PAD - E
    src_p = jnp.concatenate([src, jnp.zeros((pad,), jnp.int32)])
    dump = N + (jnp.arange(pad, dtype=jnp.int32) % (NPAD - N))
    dst_p = jnp.concatenate([dst, dump])
    zero = jnp.zeros((ROWS_PER_SUB, D), jnp.float32)

    parts1 = _agg_call(x, src_p, dst_p, zero)
    h1 = _mlp1(x, parts1[0, :N], parts1[1, :N], W1a, b1a, W1b, b1b)

    parts2 = _agg_call(h1, src_p, dst_p, zero)
    wl_pad = jnp.zeros((D, D), jnp.float32).at[:, :OUT].set(Wl)
    bl_pad = jnp.zeros((1, D), jnp.float32).at[0, :OUT].set(bl)
    out_full = _mlp2(h1, parts2[0, :N], parts2[1, :N], W2a, b2a, W2b, b2b,
                     wl_pad, bl_pad)
    return out_full[:, :OUT]


# speedup vs baseline: 2.8168x; 1.2181x over previous
import functools

import jax
import jax.numpy as jnp
from jax import lax
from jax.experimental import pallas as pl
from jax.experimental.pallas import tpu as pltpu
from jax.experimental.pallas import tpu_sc as plsc

N = 10000
D = 128
E = 320000
OUT = 2

NC = 2
NS = 16
NW = NC * NS

K = 128
NBLK = 4
BCH = 20
CHUNKS = NBLK * BCH
EPW = CHUNKS * K
E_PAD = EPW * NW
NPAD = 10112
ROWS_PER_SUB = NPAD // NS

_sc_mesh = plsc.VectorSubcoreMesh(core_axis_name="c", subcore_axis_name="s")


def _agg_body(table, idx_i, zero_hbm, out, acc, ibuf0, ibuf1,
              rows0, rows1, isem0, isem1, sem0, sem1):
    c = lax.axis_index("c")
    s = lax.axis_index("s")
    wid = s * NC + c
    pltpu.async_copy(idx_i.at[wid, 0], ibuf0, isem0)
    pltpu.sync_copy(zero_hbm, acc.at[pl.ds(s * ROWS_PER_SUB, ROWS_PER_SUB)])
    plsc.subcore_barrier()

    ibufs = (ibuf0, ibuf1)
    isems = (isem0, isem1)
    for b in range(NBLK):
        ib = ibufs[b % 2]
        pltpu.make_async_copy(idx_i.at[wid, b], ib, isems[b % 2]).wait()
        if b + 1 < NBLK:
            pltpu.async_copy(idx_i.at[wid, b + 1], ibufs[(b + 1) % 2],
                             isems[(b + 1) % 2])

        pltpu.async_copy(table.at[ib.at[0, 0]], rows0, sem0)

        def pair(j, carry):
            i0 = 2 * j
            pltpu.async_copy(table.at[ib.at[i0 + 1, 0]], rows1, sem1)
            pltpu.make_async_copy(table.at[ib.at[i0, 0]], rows0, sem0).wait()
            pltpu.sync_copy(rows0, acc.at[ib.at[i0, 1]], add=True)

            @pl.when(j < BCH // 2 - 1)
            def _():
                pltpu.async_copy(table.at[ib.at[i0 + 2, 0]], rows0, sem0)

            pltpu.make_async_copy(table.at[ib.at[i0 + 1, 0]], rows1, sem1).wait()
            pltpu.sync_copy(rows1, acc.at[ib.at[i0 + 1, 1]], add=True)
            return carry

        lax.fori_loop(0, BCH // 2, pair, 0)

    plsc.subcore_barrier()
    pltpu.sync_copy(acc.at[pl.ds(s * ROWS_PER_SUB, ROWS_PER_SUB)],
                    out.at[c, pl.ds(s * ROWS_PER_SUB, ROWS_PER_SUB)])


_agg_call = functools.partial(
    pl.kernel,
    _agg_body,
    out_type=jax.ShapeDtypeStruct((NC, NPAD, D), jnp.float32),
    mesh=_sc_mesh,
    scratch_types=[
        pltpu.VMEM_SHARED((NPAD, D), jnp.float32),
        pltpu.VMEM((BCH, 2, K), jnp.int32),
        pltpu.VMEM((BCH, 2, K), jnp.int32),
        pltpu.VMEM((K, D), jnp.float32),
        pltpu.VMEM((K, D), jnp.float32),
        pltpu.SemaphoreType.DMA,
        pltpu.SemaphoreType.DMA,
        pltpu.SemaphoreType.DMA,
        pltpu.SemaphoreType.DMA,
    ],
)()


ROWS_TC = 1000


def _mlp1_body(x_ref, a0_ref, a1_ref, wa_ref, ba_ref, wb_ref, bb_ref, o_ref):
    h = x_ref[...] + a0_ref[...] + a1_ref[...]
    t = jnp.dot(h, wa_ref[...], preferred_element_type=jnp.float32) + ba_ref[...]
    t = jnp.maximum(t, 0.0)
    u = jnp.dot(t, wb_ref[...], preferred_element_type=jnp.float32) + bb_ref[...]
    o_ref[...] = jnp.maximum(u, 0.0)


def _mlp2_body(x_ref, a0_ref, a1_ref, wa_ref, ba_ref, wb_ref, bb_ref,
               wl_ref, bl_ref, o_ref):
    h = x_ref[...] + a0_ref[...] + a1_ref[...]
    t = jnp.dot(h, wa_ref[...], preferred_element_type=jnp.float32) + ba_ref[...]
    t = jnp.maximum(t, 0.0)
    u = jnp.dot(t, wb_ref[...], preferred_element_type=jnp.float32) + bb_ref[...]
    u = jnp.maximum(u, 0.0)
    o_ref[...] = jnp.dot(u, wl_ref[...], preferred_element_type=jnp.float32) + bl_ref[...]


def _row_spec():
    return pl.BlockSpec((ROWS_TC, D), lambda i: (i, 0))


def _full_spec(shape):
    return pl.BlockSpec(shape, lambda i: (0,) * len(shape))


def _mlp1(x, a0, a1, wa, ba, wb, bb):
    return pl.pallas_call(
        _mlp1_body,
        grid=(N // ROWS_TC,),
        in_specs=[_row_spec(), _row_spec(), _row_spec(),
                  _full_spec((D, D)), _full_spec((1, D)),
                  _full_spec((D, D)), _full_spec((1, D))],
        out_specs=_row_spec(),
        out_shape=jax.ShapeDtypeStruct((N, D), jnp.float32),
    )(x, a0, a1, wa, ba.reshape(1, D), wb, bb.reshape(1, D))


def _mlp2(x, a0, a1, wa, ba, wb, bb, wl_pad, bl_pad):
    return pl.pallas_call(
        _mlp2_body,
        grid=(N // ROWS_TC,),
        in_specs=[_row_spec(), _row_spec(), _row_spec(),
                  _full_spec((D, D)), _full_spec((1, D)),
                  _full_spec((D, D)), _full_spec((1, D)),
                  _full_spec((D, D)), _full_spec((1, D))],
        out_specs=_row_spec(),
        out_shape=jax.ShapeDtypeStruct((N, D), jnp.float32),
    )(x, a0, a1, wa, ba.reshape(1, D), wb, bb.reshape(1, D), wl_pad, bl_pad)


def kernel(x, edge_index, W1a, b1a, W1b, b1b, W2a, b2a, W2b, b2b, Wl, bl):
    src = edge_index[0]
    dst = edge_index[1]
    pad = E_PAD - E
    src_p = jnp.concatenate([src, jnp.zeros((pad,), jnp.int32)])
    dump = N + (jnp.arange(pad, dtype=jnp.int32) % (NPAD - N))
    dst_p = jnp.concatenate([dst, dump])
    idx_p = jnp.stack([src_p.reshape(NW, NBLK, BCH, K),
                       dst_p.reshape(NW, NBLK, BCH, K)], axis=3)
    zero = jnp.zeros((ROWS_PER_SUB, D), jnp.float32)

    parts1 = _agg_call(x, idx_p, zero)
    h1 = _mlp1(x, parts1[0, :N], parts1[1, :N], W1a, b1a, W1b, b1b)

    parts2 = _agg_call(h1, idx_p, zero)
    wl_pad = jnp.zeros((D, D), jnp.float32).at[:, :OUT].set(Wl)
    bl_pad = jnp.zeros((1, D), jnp.float32).at[0, :OUT].set(bl)
    out_full = _mlp2(h1, parts2[0, :N], parts2[1, :N], W2a, b2a, W2b, b2b,
                     wl_pad, bl_pad)
    return out_full[:, :OUT]
